# baseline (device time: 65623 ns/iter reference)
import jax
import jax.numpy as jnp
from jax import lax
from jax.experimental import pallas as pl
from jax.experimental.pallas import tpu as pltpu

N_DEV = 8

_HALVES = ((0, 128), (128, 128))
_C_SLICES = ((0, 88), (88, 88), (176, 80))


def kernel(x, w_mat):
    m_per, k = x.shape
    _, n_per = w_mat.shape
    m_glob = m_per * N_DEV

    def body(x_ref, w_ref, out_ref, g_ref, send_sems, recv_sems):
        my = lax.axis_index("i")
        partners = (my ^ 1, my ^ 3, my ^ 4)

        def g_at(origin, row0=0, rows=m_per):
            return g_ref.at[pl.ds(origin * m_per + row0, rows), :]

        def relu_gemm(origin, row0=0, rows=m_per):
            blk = jnp.dot(g_ref[pl.ds(origin * m_per + row0, rows), :],
                          w_ref[:, :], preferred_element_type=jnp.float32)
            out_ref[pl.ds(origin * m_per + row0, rows), :] = (
                jnp.maximum(blk, 0.0))

        def copy(sem_idx, link, src_ref, dst_ref):
            return pltpu.make_async_remote_copy(
                src_ref=src_ref,
                dst_ref=dst_ref,
                send_sem=send_sems.at[sem_idx],
                recv_sem=recv_sems.at[sem_idx],
                device_id=(partners[link],),
                device_id_type=pl.DeviceIdType.MESH,
            )

        barrier_sem = pltpu.get_barrier_semaphore()
        for p in partners:
            pl.semaphore_signal(
                barrier_sem, inc=1,
                device_id=(p,), device_id_type=pl.DeviceIdType.MESH,
            )
        pl.semaphore_wait(barrier_sem, 3)

        A = {}
        for h, (r0, nr) in enumerate(_HALVES):
            for l in range(3):
                A[l, h] = copy(h * 3 + l, l,
                               x_ref.at[pl.ds(r0, nr), :], g_at(my, r0, nr))
                A[l, h].start()

        blk = jnp.dot(x_ref[:, :], w_ref[:, :],
                      preferred_element_type=jnp.float32)
        out_ref[pl.ds(my * m_per, m_per), :] = jnp.maximum(blk, 0.0)

        fwd = (my ^ 3, my ^ 4, my ^ 1)
        src_link = (1, 2, 0)
        B = {}
        for h, (r0, nr) in enumerate(_HALVES):
            for l in range(3):
                A[src_link[l], h].wait_recv()
                B[l, h] = copy(6 + h * 3 + l, l,
                               g_at(fwd[l], r0, nr), g_at(fwd[l], r0, nr))
                B[l, h].start()
            for o in (my ^ 1, my ^ 3, my ^ 4):
                relu_gemm(o, r0, nr)

        opp_src = (my ^ 7, my ^ 5, my ^ 2)
        C = {}

        def start_c(l):
            r0, nr = _C_SLICES[l]
            C[l] = copy(12 + l, l, g_at(opp_src[l], r0, nr),
                        g_at(opp_src[l], r0, nr))
            C[l].start()

        B[1, 0].wait_recv()
        start_c(0)
        B[2, 0].wait_recv()
        B[0, 0].wait_recv()
        for o in (my ^ 7, my ^ 5, my ^ 2):
            relu_gemm(o, 0, 128)
        B[2, 1].wait_recv()
        start_c(1)
        B[0, 1].wait_recv()
        start_c(2)
        B[1, 1].wait_recv()
        for o in (my ^ 7, my ^ 5, my ^ 2):
            relu_gemm(o, 128, 128)

        for l, (r0, nr) in enumerate(_C_SLICES):
            C[l].wait_recv()
            relu_gemm(my ^ 6, r0, nr)

        for d in list(A.values()) + list(B.values()) + list(C.values()):
            d.wait_send()

    return pl.pallas_call(
        body,
        out_shape=jax.ShapeDtypeStruct((m_glob, n_per), jnp.float32),
        in_specs=[
            pl.BlockSpec(memory_space=pltpu.VMEM),
            pl.BlockSpec(memory_space=pltpu.VMEM),
        ],
        out_specs=pl.BlockSpec(memory_space=pltpu.VMEM),
        scratch_shapes=[
            pltpu.VMEM((m_glob, k), jnp.float32),
            pltpu.SemaphoreType.DMA((15,)),
            pltpu.SemaphoreType.DMA((15,)),
        ],
        compiler_params=pltpu.CompilerParams(collective_id=0),
    )(x, w_mat)
